# baseline (device time: 44254 ns/iter reference)
import jax
import jax.numpy as jnp
from jax import lax
from jax.experimental import pallas as pl
from jax.experimental.pallas import tpu as pltpu

N_DEV = 4


def kernel(x):
    m, n = x.shape
    H = m // 2
    Q = H // 2
    E = Q // 2
    E2 = E // 2

    def body(x_ref, out_ref,
             rs1a_s2_ref, rs1a_own_ref, rs1b_s2_ref, rs1b_own_ref,
             rs2a_ref, rs2b_ref, send_sems, recv_sems):
        i = lax.axis_index("i")
        b0 = i & 1
        b1 = i >> 1
        p1 = i ^ 1
        p2 = 3 - i

        a_own = Q * (b0 ^ b1) + E * b1
        a_send2 = Q * (b0 ^ b1) + E * (1 - b1)
        a_own_p1 = Q * (1 - (b0 ^ b1)) + E * b1
        a_send2_p1 = Q * (1 - (b0 ^ b1)) + E * (1 - b1)
        b_own = Q * b1 + E * b0
        b_send2 = Q * b1 + E * (1 - b0)
        b_own_p2 = Q * (1 - b1) + E * (1 - b0)
        b_send2_p2 = Q * (1 - b1) + E * b0

        barrier_sem = pltpu.get_barrier_semaphore()
        for nbr in [p1, p2]:
            pl.semaphore_signal(
                barrier_sem, inc=1,
                device_id=(nbr,), device_id_type=pl.DeviceIdType.MESH,
            )
        pl.semaphore_wait(barrier_sem, 2)

        def copy(src_ref, src_off, dst_ref, dst_off, rows, sem_idx, peer):
            return pltpu.make_async_remote_copy(
                src_ref=src_ref.at[pl.ds(src_off, rows), :],
                dst_ref=dst_ref.at[pl.ds(dst_off, rows), :],
                send_sem=send_sems.at[sem_idx],
                recv_sem=recv_sems.at[sem_idx],
                device_id=(peer,),
                device_id_type=pl.DeviceIdType.MESH,
            )

        s1a_1 = copy(x_ref, a_send2_p1, rs1a_s2_ref, 0, E, 0, p1)
        s1a_2 = copy(x_ref, a_own_p1, rs1a_own_ref, 0, E, 1, p1)
        s1b_1 = copy(x_ref, H + b_send2_p2, rs1b_s2_ref, 0, E, 2, p2)
        s1b_2 = copy(x_ref, H + b_own_p2, rs1b_own_ref, 0, E, 3, p2)
        s1a_1.start()
        s1b_1.start()
        s1a_2.start()
        s1b_2.start()

        s1a_1.wait_recv()
        out_ref[pl.ds(a_send2, E2), :] = (
            x_ref[pl.ds(a_send2, E2), :] + rs1a_s2_ref[pl.ds(0, E2), :]
        )
        s2a_1 = copy(out_ref, a_send2, rs2a_ref, 0, E2, 4, p2)
        s2a_1.start()
        out_ref[pl.ds(a_send2 + E2, E2), :] = (
            x_ref[pl.ds(a_send2 + E2, E2), :]
            + rs1a_s2_ref[pl.ds(E2, E2), :]
        )
        s2a_2 = copy(out_ref, a_send2 + E2, rs2a_ref, E2, E2, 5, p2)
        s2a_2.start()

        s1b_1.wait_recv()
        out_ref[pl.ds(H + b_send2, E2), :] = (
            x_ref[pl.ds(H + b_send2, E2), :] + rs1b_s2_ref[pl.ds(0, E2), :]
        )
        s2b_1 = copy(out_ref, H + b_send2, rs2b_ref, 0, E2, 6, p1)
        s2b_1.start()
        out_ref[pl.ds(H + b_send2 + E2, E2), :] = (
            x_ref[pl.ds(H + b_send2 + E2, E2), :]
            + rs1b_s2_ref[pl.ds(E2, E2), :]
        )
        s2b_2 = copy(out_ref, H + b_send2 + E2, rs2b_ref, E2, E2, 7, p1)
        s2b_2.start()

        s1a_2.wait_recv()
        out_ref[pl.ds(a_own, E), :] = (
            x_ref[pl.ds(a_own, E), :] + rs1a_own_ref[:, :]
        )
        s1b_2.wait_recv()
        out_ref[pl.ds(H + b_own, E), :] = (
            x_ref[pl.ds(H + b_own, E), :] + rs1b_own_ref[:, :]
        )

        s2a_1.wait_recv()
        out_ref[pl.ds(a_own, E2), :] += rs2a_ref[pl.ds(0, E2), :]
        ag2a_1 = copy(out_ref, a_own, out_ref, a_own, E2, 8, p2)
        ag2a_1.start()
        s2a_2.wait_recv()
        out_ref[pl.ds(a_own + E2, E2), :] += rs2a_ref[pl.ds(E2, E2), :]
        ag2a_2 = copy(out_ref, a_own + E2, out_ref, a_own + E2, E2, 9, p2)
        ag2a_2.start()

        s2b_1.wait_recv()
        out_ref[pl.ds(H + b_own, E2), :] += rs2b_ref[pl.ds(0, E2), :]
        ag2b_1 = copy(out_ref, H + b_own, out_ref, H + b_own, E2, 10, p1)
        ag2b_1.start()
        s2b_2.wait_recv()
        out_ref[pl.ds(H + b_own + E2, E2), :] += rs2b_ref[pl.ds(E2, E2), :]
        ag2b_2 = copy(out_ref, H + b_own + E2, out_ref, H + b_own + E2,
                      E2, 11, p1)
        ag2b_2.start()

        ag1a_own = copy(out_ref, a_own, out_ref, a_own, E, 12, p1)
        ag1a_own.start()
        ag1b_own = copy(out_ref, H + b_own, out_ref, H + b_own, E, 13, p2)
        ag1b_own.start()

        ag2a_1.wait_recv()
        ag1a_oth1 = copy(out_ref, a_send2, out_ref, a_send2, E2, 14, p1)
        ag1a_oth1.start()
        ag2b_1.wait_recv()
        ag1b_oth1 = copy(out_ref, H + b_send2, out_ref, H + b_send2,
                         E2, 15, p2)
        ag1b_oth1.start()
        ag2a_2.wait_recv()
        ag1a_oth2 = copy(out_ref, a_send2 + E2, out_ref, a_send2 + E2,
                         E2, 16, p1)
        ag1a_oth2.start()
        ag2b_2.wait_recv()
        ag1b_oth2 = copy(out_ref, H + b_send2 + E2, out_ref,
                         H + b_send2 + E2, E2, 17, p2)
        ag1b_oth2.start()

        ag1a_own.wait_recv()
        ag1a_oth1.wait_recv()
        ag1a_oth2.wait_recv()
        ag1b_own.wait_recv()
        ag1b_oth1.wait_recv()
        ag1b_oth2.wait_recv()

        for r in [s1a_1, s1a_2, s1b_1, s1b_2,
                  s2a_1, s2a_2, s2b_1, s2b_2,
                  ag2a_1, ag2a_2, ag2b_1, ag2b_2,
                  ag1a_own, ag1b_own,
                  ag1a_oth1, ag1a_oth2, ag1b_oth1, ag1b_oth2]:
            r.wait_send()

    return pl.pallas_call(
        body,
        out_shape=jax.ShapeDtypeStruct((m, n), x.dtype),
        in_specs=[pl.BlockSpec(memory_space=pltpu.VMEM)],
        out_specs=pl.BlockSpec(memory_space=pltpu.VMEM),
        scratch_shapes=[
            pltpu.VMEM((E, n), x.dtype),
            pltpu.VMEM((E, n), x.dtype),
            pltpu.VMEM((E, n), x.dtype),
            pltpu.VMEM((E, n), x.dtype),
            pltpu.VMEM((E, n), x.dtype),
            pltpu.VMEM((E, n), x.dtype),
            pltpu.SemaphoreType.DMA((18,)),
            pltpu.SemaphoreType.DMA((18,)),
        ],
        compiler_params=pltpu.CompilerParams(collective_id=0),
    )(x)


# device time: 44237 ns/iter; 1.0004x vs baseline; 1.0004x over previous
import jax
import jax.numpy as jnp
from jax import lax
from jax.experimental import pallas as pl
from jax.experimental.pallas import tpu as pltpu

N_DEV = 4


def kernel(x):
    m, n = x.shape
    H = m // 2
    Q = H // 2
    E = Q // 2
    E2 = E // 2

    def body(x_ref, out_ref,
             rs1a_s2_ref, rs1a_own_ref, rs1b_s2_ref, rs1b_own_ref,
             rs2a_ref, rs2b_ref, send_sems, recv_sems):
        i = lax.axis_index("i")
        b0 = i & 1
        b1 = i >> 1
        p1 = i ^ 1
        p2 = 3 - i

        a_own = Q * (b0 ^ b1) + E * b1
        a_send2 = Q * (b0 ^ b1) + E * (1 - b1)
        a_own_p1 = Q * (1 - (b0 ^ b1)) + E * b1
        a_send2_p1 = Q * (1 - (b0 ^ b1)) + E * (1 - b1)
        b_own = Q * b1 + E * b0
        b_send2 = Q * b1 + E * (1 - b0)
        b_own_p2 = Q * (1 - b1) + E * (1 - b0)
        b_send2_p2 = Q * (1 - b1) + E * b0

        barrier_sem = pltpu.get_barrier_semaphore()
        for nbr in [p1, p2]:
            pl.semaphore_signal(
                barrier_sem, inc=1,
                device_id=(nbr,), device_id_type=pl.DeviceIdType.MESH,
            )
        pl.semaphore_wait(barrier_sem, 2)

        def copy(src_ref, src_off, dst_ref, dst_off, rows, sem_idx, peer):
            return pltpu.make_async_remote_copy(
                src_ref=src_ref.at[pl.ds(src_off, rows), :],
                dst_ref=dst_ref.at[pl.ds(dst_off, rows), :],
                send_sem=send_sems.at[sem_idx],
                recv_sem=recv_sems.at[sem_idx],
                device_id=(peer,),
                device_id_type=pl.DeviceIdType.MESH,
            )

        s1a_1 = copy(x_ref, a_send2_p1, rs1a_s2_ref, 0, E, 0, p1)
        s1a_2 = copy(x_ref, a_own_p1, rs1a_own_ref, 0, E, 1, p1)
        s1b_1 = copy(x_ref, H + b_send2_p2, rs1b_s2_ref, 0, E, 2, p2)
        s1b_2 = copy(x_ref, H + b_own_p2, rs1b_own_ref, 0, E, 3, p2)
        s1a_1.start()
        s1b_1.start()
        s1a_2.start()
        s1b_2.start()

        s1a_1.wait_recv()
        out_ref[pl.ds(a_send2, E2), :] = (
            x_ref[pl.ds(a_send2, E2), :] + rs1a_s2_ref[pl.ds(0, E2), :]
        )
        s2a_1 = copy(out_ref, a_send2, rs2a_ref, 0, E2, 4, p2)
        s2a_1.start()
        out_ref[pl.ds(a_send2 + E2, E2), :] = (
            x_ref[pl.ds(a_send2 + E2, E2), :]
            + rs1a_s2_ref[pl.ds(E2, E2), :]
        )
        s2a_2 = copy(out_ref, a_send2 + E2, rs2a_ref, E2, E2, 5, p2)
        s2a_2.start()

        s1b_1.wait_recv()
        out_ref[pl.ds(H + b_send2, E2), :] = (
            x_ref[pl.ds(H + b_send2, E2), :] + rs1b_s2_ref[pl.ds(0, E2), :]
        )
        s2b_1 = copy(out_ref, H + b_send2, rs2b_ref, 0, E2, 6, p1)
        s2b_1.start()
        out_ref[pl.ds(H + b_send2 + E2, E2), :] = (
            x_ref[pl.ds(H + b_send2 + E2, E2), :]
            + rs1b_s2_ref[pl.ds(E2, E2), :]
        )
        s2b_2 = copy(out_ref, H + b_send2 + E2, rs2b_ref, E2, E2, 7, p1)
        s2b_2.start()

        s1a_2.wait_recv()
        out_ref[pl.ds(a_own, E), :] = (
            x_ref[pl.ds(a_own, E), :] + rs1a_own_ref[:, :]
        )
        s1b_2.wait_recv()
        out_ref[pl.ds(H + b_own, E), :] = (
            x_ref[pl.ds(H + b_own, E), :] + rs1b_own_ref[:, :]
        )

        s2a_1.wait_recv()
        out_ref[pl.ds(a_own, E2), :] += rs2a_ref[pl.ds(0, E2), :]
        ag2a_1 = copy(out_ref, a_own, out_ref, a_own, E2, 8, p2)
        ag2a_1.start()
        s2a_2.wait_recv()
        out_ref[pl.ds(a_own + E2, E2), :] += rs2a_ref[pl.ds(E2, E2), :]
        ag2a_2 = copy(out_ref, a_own + E2, out_ref, a_own + E2, E2, 9, p2)
        ag2a_2.start()

        s2b_1.wait_recv()
        out_ref[pl.ds(H + b_own, E2), :] += rs2b_ref[pl.ds(0, E2), :]
        ag2b_1 = copy(out_ref, H + b_own, out_ref, H + b_own, E2, 10, p1)
        ag2b_1.start()
        s2b_2.wait_recv()
        out_ref[pl.ds(H + b_own + E2, E2), :] += rs2b_ref[pl.ds(E2, E2), :]
        ag2b_2 = copy(out_ref, H + b_own + E2, out_ref, H + b_own + E2,
                      E2, 11, p1)
        ag2b_2.start()

        ag1a_own = copy(out_ref, a_own, out_ref, a_own, E, 12, p1)
        ag1a_own.start()
        ag1b_own = copy(out_ref, H + b_own, out_ref, H + b_own, E, 13, p2)
        ag1b_own.start()

        ag2a_1.wait_recv()
        ag2a_2.wait_recv()
        ag1a_oth = copy(out_ref, a_send2, out_ref, a_send2, E, 14, p1)
        ag1a_oth.start()

        ag2b_1.wait_recv()
        ag2b_2.wait_recv()
        ag1b_oth = copy(out_ref, H + b_send2, out_ref, H + b_send2,
                        E, 15, p2)
        ag1b_oth.start()

        ag1a_own.wait_recv()
        ag1a_oth.wait_recv()
        ag1b_own.wait_recv()
        ag1b_oth.wait_recv()

        for r in [s1a_1, s1a_2, s1b_1, s1b_2,
                  s2a_1, s2a_2, s2b_1, s2b_2,
                  ag2a_1, ag2a_2, ag2b_1, ag2b_2,
                  ag1a_own, ag1b_own, ag1a_oth, ag1b_oth]:
            r.wait_send()

    return pl.pallas_call(
        body,
        out_shape=jax.ShapeDtypeStruct((m, n), x.dtype),
        in_specs=[pl.BlockSpec(memory_space=pltpu.VMEM)],
        out_specs=pl.BlockSpec(memory_space=pltpu.VMEM),
        scratch_shapes=[
            pltpu.VMEM((E, n), x.dtype),
            pltpu.VMEM((E, n), x.dtype),
            pltpu.VMEM((E, n), x.dtype),
            pltpu.VMEM((E, n), x.dtype),
            pltpu.VMEM((E, n), x.dtype),
            pltpu.VMEM((E, n), x.dtype),
            pltpu.SemaphoreType.DMA((16,)),
            pltpu.SemaphoreType.DMA((16,)),
        ],
        compiler_params=pltpu.CompilerParams(collective_id=0),
    )(x)
